# baseline (speedup 1.0000x reference)
import jax
import jax.numpy as jnp
from jax import lax
from jax.experimental import pallas as pl
from jax.experimental.pallas import tpu as pltpu
from jax.experimental.pallas import tpu_sc as plsc

B = 16384
F = 26
V = 100000
D = 32
DP = 128

NC = 2
NS = 16
L = 16
NW = NC * NS
RPW = B // NW
RU = 256
CHUNK = 128
NBUF = 3


def _body(tbl_ref, inp_ref, cardb_ref, *refs):
    outs = refs[:F]
    idx_v, card_v = refs[F:F + 2]
    gidx = refs[F + 2:F + 2 + F]
    rows = refs[F + 2 + F:F + 2 + F + NBUF]
    gsems = refs[F + 2 + F + NBUF:F + 2 + F + 2 * NBUF]
    osems = refs[F + 2 + F + 2 * NBUF:]
    wid = lax.axis_index("s") * NC + lax.axis_index("c")
    base = wid * RPW

    pltpu.sync_copy(inp_ref.at[:, pl.ds(base, RPW)], idx_v)
    pltpu.sync_copy(cardb_ref, card_v)

    for f in range(F):
        cvec = card_v[pl.ds(f * L, L)]

        def compute(i, _, cvec=cvec, f=f):
            h = idx_v[f, pl.ds(i * L, L)] + 1
            h = jnp.where(h >= cvec, h - cvec, h)
            gidx[f][pl.ds(i * L, L)] = h
            return 0

        lax.fori_loop(0, RPW // L, compute, 0)

    units = [(f, h) for f in range(F) for h in range(RPW // RU)]
    gcopies = {}
    ocopies = {}
    for u in range(len(units) + 1):
        if u < len(units):
            f, h = units[u]
            b = u % NBUF
            if u >= NBUF:
                ocopies.pop(u - NBUF).wait()
            gcopies[u] = [
                pltpu.async_copy(
                    tbl_ref.at[f].at[gidx[f].at[pl.ds(h * RU + j * CHUNK, CHUNK)]],
                    rows[b].at[pl.ds(j * CHUNK, CHUNK)],
                    gsems[b],
                )
                for j in range(RU // CHUNK)
            ]
        if u >= 1:
            g = u - 1
            gf, gh = units[g]
            for c in gcopies.pop(g):
                c.wait()
            ocopies[g] = pltpu.async_copy(
                rows[g % NBUF],
                outs[gf].at[pl.ds(base + gh * RU, RU)],
                osems[g % NBUF],
            )
    for g in sorted(ocopies):
        ocopies.pop(g).wait()


@jax.jit
def _run(tbl, inp_t, card_b):
    mesh = plsc.VectorSubcoreMesh(core_axis_name="c", subcore_axis_name="s")
    fn = pl.kernel(
        _body,
        out_type=tuple(
            jax.ShapeDtypeStruct((B, D), jnp.float32) for _ in range(F)
        ),
        mesh=mesh,
        scratch_types=(
            [
                pltpu.VMEM((F, RPW), jnp.int32),
                pltpu.VMEM((F * L,), jnp.int32),
            ]
            + [pltpu.VMEM((RPW,), jnp.int32) for _ in range(F)]

            + [pltpu.VMEM((RU, D), jnp.float32) for _ in range(NBUF)]
            + [pltpu.SemaphoreType.DMA for _ in range(2 * NBUF)]
        ),
        compiler_params=pltpu.CompilerParams(use_tc_tiling_on_sc=False),
    )
    return fn(tbl, inp_t, card_b)


def kernel(inputs, tables, cardinality):
    inp_t = inputs.T
    card_b = jnp.broadcast_to(
        cardinality.astype(jnp.int32)[:, None], (F, L)
    ).reshape(F * L)
    return tuple(_run(tables, inp_t, card_b))
